# baseline (speedup 1.0000x reference)
import functools

import jax
import jax.numpy as jnp
from jax import lax
from jax.experimental import pallas as pl


def _pool_body(x_ref, o_ref, *, inv_hw):
    s = jnp.sum(x_ref[...], axis=(2, 3)) * inv_hw
    o_ref[...] = s[:, None, None, :]


def _scale_body(mean_ref, wfc_ref, wgat_ref, o_ref):
    y = lax.dot_general(mean_ref[...], wfc_ref[...], (((1,), (1,)), ((), ())),
                        preferred_element_type=jnp.float32)
    y = jnp.maximum(y, 0.0)
    s = lax.dot_general(y, wgat_ref[...], (((1,), (1,)), ((), ())),
                        preferred_element_type=jnp.float32)
    o_ref[...] = s


def _mul_body(x_ref, s_ref, o_ref):
    o_ref[...] = x_ref[...] * s_ref[...]


def kernel(x, W_fc, W_gat):
    b, c, H, Wd = x.shape
    CB = 32
    grid = (b, c // CB)

    means = pl.pallas_call(
        functools.partial(_pool_body, inv_hw=1.0 / (H * Wd)),
        grid=grid,
        in_specs=[pl.BlockSpec((1, CB, H, Wd), lambda i, j: (i, j, 0, 0))],
        out_specs=pl.BlockSpec((1, 1, 1, CB), lambda i, j: (i, j, 0, 0)),
        out_shape=jax.ShapeDtypeStruct((b, c // CB, 1, CB), jnp.float32),
    )(x)

    scale = pl.pallas_call(
        _scale_body,
        out_shape=jax.ShapeDtypeStruct((b, c), jnp.float32),
    )(means.reshape(b, c), W_fc, W_gat)

    out = pl.pallas_call(
        _mul_body,
        grid=grid,
        in_specs=[
            pl.BlockSpec((1, CB, H, Wd), lambda i, j: (i, j, 0, 0)),
            pl.BlockSpec((1, CB, 1, 1), lambda i, j: (i, j, 0, 0)),
        ],
        out_specs=pl.BlockSpec((1, CB, H, Wd), lambda i, j: (i, j, 0, 0)),
        out_shape=jax.ShapeDtypeStruct(x.shape, jnp.float32),
    )(x, scale.reshape(b, c, 1, 1))

    return out


# speedup vs baseline: 3.3154x; 3.3154x over previous
import functools

import jax
import jax.numpy as jnp
from jax import lax
from jax.experimental import pallas as pl


def _pool_body(x_ref, o_ref, *, inv_hw):
    j = pl.program_id(1)
    part = jnp.sum(x_ref[...], axis=(1, 2)) * inv_hw

    @pl.when(j == 0)
    def _():
        o_ref[...] = part[:, None, :]

    @pl.when(j != 0)
    def _():
        o_ref[...] += part[:, None, :]


def _scale_body(mean_ref, wfc_ref, wgat_ref, o_ref):
    y = lax.dot_general(mean_ref[...], wfc_ref[...], (((1,), (1,)), ((), ())),
                        preferred_element_type=jnp.float32)
    y = jnp.maximum(y, 0.0)
    s = lax.dot_general(y, wgat_ref[...], (((1,), (1,)), ((), ())),
                        preferred_element_type=jnp.float32)
    o_ref[...] = s[:, None, :]


def _mul_body(x_ref, s_ref, o_ref):
    o_ref[...] = x_ref[...] * s_ref[...][:, None, :, :]


def kernel(x, W_fc, W_gat):
    b, c, H, Wd = x.shape
    xt = jnp.transpose(x, (0, 2, 3, 1))
    HB = 16
    grid = (b, H // HB)

    means = pl.pallas_call(
        functools.partial(_pool_body, inv_hw=1.0 / (H * Wd)),
        grid=grid,
        in_specs=[pl.BlockSpec((1, HB, Wd, c), lambda i, j: (i, j, 0, 0))],
        out_specs=pl.BlockSpec((1, 1, c), lambda i, j: (i, 0, 0)),
        out_shape=jax.ShapeDtypeStruct((b, 1, c), jnp.float32),
    )(xt)

    scale = pl.pallas_call(
        _scale_body,
        out_shape=jax.ShapeDtypeStruct((b, 1, c), jnp.float32),
    )(means.reshape(b, c), W_fc, W_gat)

    out = pl.pallas_call(
        _mul_body,
        grid=grid,
        in_specs=[
            pl.BlockSpec((1, HB, Wd, c), lambda i, j: (i, j, 0, 0)),
            pl.BlockSpec((1, 1, c), lambda i, j: (i, 0, 0)),
        ],
        out_specs=pl.BlockSpec((1, HB, Wd, c), lambda i, j: (i, j, 0, 0)),
        out_shape=jax.ShapeDtypeStruct((b, H, Wd, c), jnp.float32),
    )(xt, scale)

    return jnp.transpose(out, (0, 3, 1, 2))


# speedup vs baseline: 3.3949x; 1.0240x over previous
import functools

import jax
import jax.numpy as jnp
from jax import lax
from jax.experimental import pallas as pl
from jax.experimental.pallas import tpu as pltpu


def _pool_body(x_ref, o_ref, *, inv_hw):
    j = pl.program_id(1)
    part = jnp.sum(x_ref[...], axis=(1, 2)) * inv_hw

    @pl.when(j == 0)
    def _():
        o_ref[...] = part[:, None, :]

    @pl.when(j != 0)
    def _():
        o_ref[...] += part[:, None, :]


def _mul_body(x_ref, mean_ref, wfc_ref, wgat_ref, o_ref, s_ref):
    i = pl.program_id(0)
    j = pl.program_id(1)

    @pl.when(j == 0)
    def _():
        mean_i = mean_ref[i]
        y = lax.dot_general(mean_i, wfc_ref[...], (((1,), (1,)), ((), ())),
                            preferred_element_type=jnp.float32)
        y = jnp.maximum(y, 0.0)
        s = lax.dot_general(y, wgat_ref[...], (((1,), (1,)), ((), ())),
                            preferred_element_type=jnp.float32)
        s_ref[...] = s

    o_ref[...] = x_ref[...] * s_ref[...][None, :, None, :]


def kernel(x, W_fc, W_gat):
    b, c, H, Wd = x.shape
    xt = jnp.transpose(x, (0, 2, 3, 1))
    HB = 16
    grid = (b, H // HB)

    means = pl.pallas_call(
        functools.partial(_pool_body, inv_hw=1.0 / (H * Wd)),
        grid=grid,
        in_specs=[pl.BlockSpec((1, HB, Wd, c), lambda i, j: (i, j, 0, 0))],
        out_specs=pl.BlockSpec((1, 1, c), lambda i, j: (i, 0, 0)),
        out_shape=jax.ShapeDtypeStruct((b, 1, c), jnp.float32),
    )(xt)

    out = pl.pallas_call(
        _mul_body,
        grid=grid,
        in_specs=[
            pl.BlockSpec((1, HB, Wd, c), lambda i, j: (i, j, 0, 0)),
            pl.BlockSpec((b, 1, c), lambda i, j: (0, 0, 0)),
            pl.BlockSpec(W_fc.shape, lambda i, j: (0, 0)),
            pl.BlockSpec(W_gat.shape, lambda i, j: (0, 0)),
        ],
        out_specs=pl.BlockSpec((1, HB, Wd, c), lambda i, j: (i, j, 0, 0)),
        out_shape=jax.ShapeDtypeStruct((b, H, Wd, c), jnp.float32),
        scratch_shapes=[pltpu.VMEM((1, c), jnp.float32)],
    )(xt, means, W_fc, W_gat)

    return jnp.transpose(out, (0, 3, 1, 2))
